# baseline (speedup 1.0000x reference)
import jax
import jax.numpy as jnp
from jax.experimental import pallas as pl

_B = 4
_T = 2048
_D = 4096
_STRIDE = 2.0
_WIDTH = 30.0
_T_SEC = _T * _STRIDE
_TT = 256


def _logits_body(x_ref, qt_ref, out_ref):
    x = x_ref[...].reshape(_B * _TT, _D).astype(jnp.bfloat16)
    qt = qt_ref[...].astype(jnp.bfloat16)
    m = jax.lax.dot_general(x, qt, (((1,), (0,)), ((), ())),
                            preferred_element_type=jnp.float32)
    row_b = jax.lax.broadcasted_iota(jnp.int32, (_B * _TT, _B), 0) // _TT
    col = jax.lax.broadcasted_iota(jnp.int32, (_B * _TT, _B), 1)
    sel = (row_b == col).astype(jnp.float32)
    out_ref[...] = jnp.sum(m * sel, axis=1, keepdims=True)[None]


def _post_body(logits_ref, seg_ref, loss_ref, lo_ref, hi_ref, scores_ref):
    x = logits_ref[...]
    seg = jnp.round(seg_ref[...])
    tf = jax.lax.broadcasted_iota(jnp.int32, (_B, _T), 1).astype(jnp.float32)
    gt = ((tf >= seg[:, 0:1]) & (tf < seg[:, 1:2])).astype(jnp.float32)

    p = jax.nn.sigmoid(x)
    ce = jnp.maximum(x, 0.0) - x * gt + jnp.log1p(jnp.exp(-jnp.abs(x)))
    p_t = p * gt + (1.0 - p) * (1.0 - gt)
    foc = ce * (1.0 - p_t) ** 2
    a_t = 0.25 * gt + 0.75 * (1.0 - gt)
    loss_ref[...] = jnp.mean(a_t * foc).reshape(1, 1)

    ti = jax.lax.broadcasted_iota(jnp.int32, (_B, _T), 1)
    radius = int(_WIDTH / _STRIDE)
    scores = x
    ms, cs = [], []
    for _ in range(5):
        m = jnp.max(scores, axis=1, keepdims=True)
        i = jnp.min(jnp.where(scores == m, ti, _T), axis=1, keepdims=True)
        ms.append(m)
        cs.append(i.astype(jnp.float32) * _STRIDE)
        scores = jnp.where(jnp.abs(ti - i) < radius, -jnp.inf, scores)
    c = jnp.concatenate(cs, axis=1)
    scores_ref[...] = jnp.concatenate(ms, axis=1)
    lo_ref[...] = jnp.clip(c - _WIDTH / 2.0, 0.0, _T_SEC)
    hi_ref[...] = jnp.clip(c + _WIDTH / 2.0, 0.0, _T_SEC)


def kernel(z_ctx, m_ctx, z_q, gt_segment):
    del m_ctx
    qt = z_q.T
    tiles = pl.pallas_call(
        _logits_body,
        grid=(_T // _TT,),
        in_specs=[
            pl.BlockSpec((_B, _TT, _D), lambda t: (0, t, 0)),
            pl.BlockSpec((_D, _B), lambda t: (0, 0)),
        ],
        out_specs=pl.BlockSpec((1, _B * _TT, 1), lambda t: (t, 0, 0)),
        out_shape=jax.ShapeDtypeStruct((_T // _TT, _B * _TT, 1), jnp.float32),
    )(z_ctx, qt)

    logits = tiles.reshape(_T // _TT, _B, _TT).transpose(1, 0, 2).reshape(_B, _T)

    loss, lo, hi, scores = pl.pallas_call(
        _post_body,
        out_shape=(
            jax.ShapeDtypeStruct((1, 1), jnp.float32),
            jax.ShapeDtypeStruct((_B, 5), jnp.float32),
            jax.ShapeDtypeStruct((_B, 5), jnp.float32),
            jax.ShapeDtypeStruct((_B, 5), jnp.float32),
        ),
    )(logits, gt_segment)

    segments = jnp.stack([lo, hi], axis=-1)
    return (loss[0, 0], logits, segments, scores)


# speedup vs baseline: 1.2038x; 1.2038x over previous
import jax
import jax.numpy as jnp
from jax.experimental import pallas as pl

_B = 4
_T = 2048
_D = 4096
_STRIDE = 2.0
_WIDTH = 30.0
_T_SEC = _T * _STRIDE
_TT = 256


def _body(x_ref, q_ref, seg_ref, loss_ref, logits_ref, lo_ref, hi_ref,
          scores_ref):
    t = pl.program_id(0)
    xb = x_ref[...].reshape(_B * _TT, _D).astype(jnp.bfloat16)
    qb = q_ref[...].astype(jnp.bfloat16)
    m4 = jax.lax.dot_general(qb, xb, (((1,), (1,)), ((), ())),
                             preferred_element_type=jnp.float32)
    m4r = m4.reshape(_B, _B, _TT)
    bi = jax.lax.broadcasted_iota(jnp.int32, (_B, _B, _TT), 0)
    gi = jax.lax.broadcasted_iota(jnp.int32, (_B, _B, _TT), 1)
    blk = jnp.sum(m4r * (bi == gi).astype(jnp.float32), axis=1)
    logits_ref[:, pl.ds(t * _TT, _TT)] = blk

    @pl.when(t == _T // _TT - 1)
    def _epilogue():
        x = logits_ref[...]
        seg = jnp.round(seg_ref[...])
        tf = jax.lax.broadcasted_iota(jnp.int32, (_B, _T), 1).astype(jnp.float32)
        gt = ((tf >= seg[:, 0:1]) & (tf < seg[:, 1:2])).astype(jnp.float32)

        p = jax.nn.sigmoid(x)
        ce = jnp.maximum(x, 0.0) - x * gt + jnp.log1p(jnp.exp(-jnp.abs(x)))
        p_t = p * gt + (1.0 - p) * (1.0 - gt)
        foc = ce * (1.0 - p_t) ** 2
        a_t = 0.25 * gt + 0.75 * (1.0 - gt)
        loss_ref[...] = jnp.mean(a_t * foc).reshape(1, 1)

        ti = jax.lax.broadcasted_iota(jnp.int32, (_B, _T), 1)
        radius = int(_WIDTH / _STRIDE)
        scores = x
        ms, cs = [], []
        for _ in range(5):
            m = jnp.max(scores, axis=1, keepdims=True)
            i = jnp.min(jnp.where(scores == m, ti, _T), axis=1, keepdims=True)
            ms.append(m)
            cs.append(i.astype(jnp.float32) * _STRIDE)
            scores = jnp.where(jnp.abs(ti - i) < radius, -jnp.inf, scores)
        c = jnp.concatenate(cs, axis=1)
        scores_ref[...] = jnp.concatenate(ms, axis=1)
        lo_ref[...] = jnp.clip(c - _WIDTH / 2.0, 0.0, _T_SEC)
        hi_ref[...] = jnp.clip(c + _WIDTH / 2.0, 0.0, _T_SEC)


def kernel(z_ctx, m_ctx, z_q, gt_segment):
    del m_ctx
    loss, logits, lo, hi, scores = pl.pallas_call(
        _body,
        grid=(_T // _TT,),
        in_specs=[
            pl.BlockSpec((_B, _TT, _D), lambda t: (0, t, 0)),
            pl.BlockSpec((_B, _D), lambda t: (0, 0)),
            pl.BlockSpec((_B, 2), lambda t: (0, 0)),
        ],
        out_specs=(
            pl.BlockSpec((1, 1), lambda t: (0, 0)),
            pl.BlockSpec((_B, _T), lambda t: (0, 0)),
            pl.BlockSpec((_B, 5), lambda t: (0, 0)),
            pl.BlockSpec((_B, 5), lambda t: (0, 0)),
            pl.BlockSpec((_B, 5), lambda t: (0, 0)),
        ),
        out_shape=(
            jax.ShapeDtypeStruct((1, 1), jnp.float32),
            jax.ShapeDtypeStruct((_B, _T), jnp.float32),
            jax.ShapeDtypeStruct((_B, 5), jnp.float32),
            jax.ShapeDtypeStruct((_B, 5), jnp.float32),
            jax.ShapeDtypeStruct((_B, 5), jnp.float32),
        ),
    )(z_ctx, z_q, gt_segment)

    segments = jnp.stack([lo, hi], axis=-1)
    return (loss[0, 0], logits, segments, scores)


# speedup vs baseline: 1.2502x; 1.0385x over previous
import jax
import jax.numpy as jnp
from jax.experimental import pallas as pl

_B = 4
_T = 2048
_D = 4096
_STRIDE = 2.0
_WIDTH = 30.0
_T_SEC = _T * _STRIDE
_TT = 128


def _body(x_ref, q_ref, seg_ref, loss_ref, logits_ref, segs_ref, scores_ref):
    t = pl.program_id(0)
    xb = x_ref[...].reshape(_B * _TT, _D).astype(jnp.bfloat16)
    qb = q_ref[...].astype(jnp.bfloat16)
    m4 = jax.lax.dot_general(qb, xb, (((1,), (1,)), ((), ())),
                             preferred_element_type=jnp.float32)
    m4r = m4.reshape(_B, _B, _TT)
    bi = jax.lax.broadcasted_iota(jnp.int32, (_B, _B, _TT), 0)
    gi = jax.lax.broadcasted_iota(jnp.int32, (_B, _B, _TT), 1)
    blk = jnp.sum(m4r * (bi == gi).astype(jnp.float32), axis=1)
    logits_ref[:, pl.ds(t * _TT, _TT)] = blk

    @pl.when(t == _T // _TT - 1)
    def _epilogue():
        x = logits_ref[...]
        seg = jnp.round(seg_ref[...])
        tf = jax.lax.broadcasted_iota(jnp.int32, (_B, _T), 1).astype(jnp.float32)
        gt = ((tf >= seg[:, 0:1]) & (tf < seg[:, 1:2])).astype(jnp.float32)

        p = jax.nn.sigmoid(x)
        ce = jnp.maximum(x, 0.0) - x * gt + jnp.log1p(jnp.exp(-jnp.abs(x)))
        p_t = p * gt + (1.0 - p) * (1.0 - gt)
        foc = ce * (1.0 - p_t) ** 2
        a_t = 0.25 * gt + 0.75 * (1.0 - gt)
        loss_ref[...] = jnp.mean(a_t * foc).reshape(1, 1)

        ti = jax.lax.broadcasted_iota(jnp.int32, (_B, _T), 1)
        radius = int(_WIDTH / _STRIDE)
        scores = x
        ms, cs = [], []
        for _ in range(5):
            m = jnp.max(scores, axis=1, keepdims=True)
            i = jnp.min(jnp.where(scores == m, ti, _T), axis=1, keepdims=True)
            ms.append(m)
            cs.append(i.astype(jnp.float32) * _STRIDE)
            scores = jnp.where(jnp.abs(ti - i) < radius, -jnp.inf, scores)
        c = jnp.concatenate(cs, axis=1)
        scores_ref[...] = jnp.concatenate(ms, axis=1)
        lo = jnp.clip(c - _WIDTH / 2.0, 0.0, _T_SEC)
        hi = jnp.clip(c + _WIDTH / 2.0, 0.0, _T_SEC)
        segs_ref[...] = jnp.stack([lo, hi], axis=-1)


def kernel(z_ctx, m_ctx, z_q, gt_segment):
    del m_ctx
    loss, logits, segments, scores = pl.pallas_call(
        _body,
        grid=(_T // _TT,),
        in_specs=[
            pl.BlockSpec((_B, _TT, _D), lambda t: (0, t, 0)),
            pl.BlockSpec((_B, _D), lambda t: (0, 0)),
            pl.BlockSpec((_B, 2), lambda t: (0, 0)),
        ],
        out_specs=(
            pl.BlockSpec((1, 1), lambda t: (0, 0)),
            pl.BlockSpec((_B, _T), lambda t: (0, 0)),
            pl.BlockSpec((_B, 5, 2), lambda t: (0, 0, 0)),
            pl.BlockSpec((_B, 5), lambda t: (0, 0)),
        ),
        out_shape=(
            jax.ShapeDtypeStruct((1, 1), jnp.float32),
            jax.ShapeDtypeStruct((_B, _T), jnp.float32),
            jax.ShapeDtypeStruct((_B, 5, 2), jnp.float32),
            jax.ShapeDtypeStruct((_B, 5), jnp.float32),
        ),
    )(z_ctx, z_q, gt_segment)

    return (loss[0, 0], logits, segments, scores)
